# baseline (speedup 1.0000x reference)
import functools

import jax
import jax.numpy as jnp
from jax import lax
from jax.experimental import pallas as pl
from jax.experimental.pallas import tpu as pltpu
from jax.experimental.pallas import tpu_sc as plsc

_N = 10000
_E = 320000
_H = 128
_NC = 2
_NS = 16
_NW = _NC * _NS
_K = 128
_NCHUNK = _E // _K
_CPW = _NCHUNK // _NW
_NXTRA = _NCHUNK - _CPW * _NW
_RZ = 632
_BM = 1000
_BN = 1000


def _zero_acc(zb, acc, sid):
    z16 = jnp.zeros((16,), jnp.float32)
    ncv = _H // 16

    def _zb_fill(n, _):
        zb[n // ncv, pl.ds((n % ncv) * 16, 16)] = z16
        return 0
    lax.fori_loop(0, 8 * ncv, _zb_fill, 0)

    @pl.when(sid < _NS - 1)
    def _():
        def _zacc(t, _):
            pltpu.sync_copy(zb, acc.at[pl.ds(sid * _RZ + t * 8, 8)])
            return 0
        lax.fori_loop(0, _RZ // 8, _zacc, 0)

    @pl.when(sid == _NS - 1)
    def _():
        def _zacc(t, _):
            pltpu.sync_copy(zb, acc.at[pl.ds((_NS - 1) * _RZ + t * 8, 8)])
            return 0
        lax.fori_loop(0, (_N - (_NS - 1) * _RZ) // 8, _zacc, 0)


def _writeback(acc, out_p, cid, sid):
    @pl.when(sid < _NS - 1)
    def _():
        r0 = sid * _RZ
        pltpu.sync_copy(acc.at[pl.ds(r0, _RZ)],
                        out_p.at[cid].at[pl.ds(r0, _RZ)])

    @pl.when(sid == _NS - 1)
    def _():
        r0 = (_NS - 1) * _RZ
        pltpu.sync_copy(acc.at[pl.ds(r0, _N - r0)],
                        out_p.at[cid].at[pl.ds(r0, _N - r0)])


def _worker_chunks(cid, sid):
    wid = cid * _NS + sid
    base = wid * _CPW + jnp.minimum(wid, _NXTRA)
    n_my = _CPW + jnp.where(wid < _NXTRA, 1, 0)
    return base, n_my


def _agg_body(h, src2, dst2, out_p, is0, is1, is2, id0, id1, id2,
              rows0, rows1, rows2, zb, acc,
              isem0, isem1, isem2, dsem0, dsem1, dsem2,
              gsem0, gsem1, gsem2, ssem0, ssem1, ssem2):
    cid = lax.axis_index("c")
    sid = lax.axis_index("s")
    base, n_my = _worker_chunks(cid, sid)
    isb = [is0, is1, is2]
    idb = [id0, id1, id2]
    rows = [rows0, rows1, rows2]
    isem = [isem0, isem1, isem2]
    dsem = [dsem0, dsem1, dsem2]
    gsem = [gsem0, gsem1, gsem2]
    ssem = [ssem0, ssem1, ssem2]

    def _load_idx(j, b):
        pltpu.async_copy(src2.at[pl.ds((base + j) * _K, _K)], isb[b], isem[b])
        pltpu.async_copy(dst2.at[pl.ds((base + j) * _K, _K)], idb[b], dsem[b])

    def _wait_idx(b):
        pltpu.make_async_copy(src2.at[pl.ds(0, _K)], isb[b], isem[b]).wait()
        pltpu.make_async_copy(dst2.at[pl.ds(0, _K)], idb[b], dsem[b]).wait()

    def _gather(b):
        pltpu.async_copy(h.at[isb[b]], rows[b], gsem[b])

    def _wait_gather(b):
        pltpu.make_async_copy(h.at[isb[b]], rows[b], gsem[b]).wait()

    def _scatter(b):
        pltpu.async_copy(rows[b], acc.at[idb[b]], ssem[b], add=True).wait()

    _zero_acc(zb, acc, sid)
    plsc.subcore_barrier()

    _load_idx(0, 0)
    _load_idx(1, 1)
    _wait_idx(0)
    _gather(0)

    def _pair(p, _):
        j0 = 2 * p
        _wait_idx(1)
        _gather(1)
        _wait_gather(0)
        _scatter(0)

        @pl.when(j0 + 2 < n_my)
        def _():
            _load_idx(j0 + 2, 0)
            _wait_idx(0)
            _gather(0)
        _wait_gather(1)
        _scatter(1)

        @pl.when(j0 + 3 < n_my)
        def _():
            _load_idx(j0 + 3, 1)
        return 0
    lax.fori_loop(0, _CPW // 2, _pair, 0)

    @pl.when(n_my > _CPW)
    def _():
        _wait_gather(0)
        _scatter(0)

    plsc.subcore_barrier()
    _writeback(acc, out_p, cid, sid)


def _agg(h, src2, dst2):
    mesh = plsc.VectorSubcoreMesh(core_axis_name="c", subcore_axis_name="s")
    scratch = (
        [pltpu.VMEM((_K,), jnp.int32)] * 6
        + [pltpu.VMEM((_K, _H), jnp.float32)] * 3
        + [pltpu.VMEM((8, _H), jnp.float32),
           pltpu.VMEM_SHARED((_N, _H), jnp.float32)]
        + [pltpu.SemaphoreType.DMA] * 12
    )
    return pl.kernel(
        _agg_body,
        out_type=jax.ShapeDtypeStruct((_NC, _N, _H), jnp.float32),
        mesh=mesh,
        scratch_types=scratch,
        name="sc_agg",
    )(h, src2, dst2)


def _deg_body(dst2, out_p, id0, id1, id2, ones_rows, zb, acc,
              dsem0, dsem1, dsem2, ss0, ss1, ss2):
    cid = lax.axis_index("c")
    sid = lax.axis_index("s")
    base, n_my = _worker_chunks(cid, sid)
    idb = [id0, id1, id2]
    dsem = [dsem0, dsem1, dsem2]
    ssem = [ss0, ss1, ss2]

    one16 = jnp.ones((16,), jnp.float32)
    ncv = _H // 16

    def _ones_fill(n, _):
        ones_rows[n // ncv, pl.ds((n % ncv) * 16, 16)] = one16
        return 0
    lax.fori_loop(0, _K * ncv, _ones_fill, 0)

    def _load_idx(j, b):
        pltpu.async_copy(dst2.at[pl.ds((base + j) * _K, _K)], idb[b], dsem[b])

    def _wait_idx(b):
        pltpu.make_async_copy(dst2.at[pl.ds(0, _K)], idb[b], dsem[b]).wait()

    def _scatter(b):
        pltpu.async_copy(ones_rows, acc.at[idb[b]], ssem[b], add=True)

    def _wait_scatter(b):
        pltpu.make_async_copy(ones_rows, acc.at[idb[b]], ssem[b]).wait()

    _zero_acc(zb, acc, sid)
    plsc.subcore_barrier()

    _load_idx(0, 0)
    _wait_idx(0)
    _scatter(0)
    _load_idx(1, 1)
    _wait_idx(1)
    _scatter(1)
    _load_idx(2, 2)

    def _triple(t, _):
        for s in range(3):
            j = 2 + 3 * t + s
            b = (2 + s) % 3
            bn = s % 3
            _wait_scatter(bn)
            _load_idx(j + 1, bn)
            _wait_idx(b)
            _scatter(b)
        return 0
    lax.fori_loop(0, 25, _triple, 0)

    _wait_scatter(0)
    _wait_idx(2)
    _scatter(2)
    _wait_scatter(1)
    _wait_scatter(2)

    @pl.when(n_my > _CPW)
    def _():
        _load_idx(_CPW, 0)
        _wait_idx(0)
        _scatter(0)
        _wait_scatter(0)

    plsc.subcore_barrier()
    _writeback(acc, out_p, cid, sid)


def _deg(dst2):
    mesh = plsc.VectorSubcoreMesh(core_axis_name="c", subcore_axis_name="s")
    return pl.kernel(
        _deg_body,
        out_type=jax.ShapeDtypeStruct((_NC, _N, _H), jnp.float32),
        mesh=mesh,
        scratch_types=[pltpu.VMEM((_K,), jnp.int32)] * 3 + [
            pltpu.VMEM((_K, _H), jnp.float32),
            pltpu.VMEM((8, _H), jnp.float32),
            pltpu.VMEM_SHARED((_N, _H), jnp.float32),
        ] + [pltpu.SemaphoreType.DMA] * 6,
        name="sc_deg",
    )(dst2)


def _tc0_body(f_ref, w_ref, b_ref, o_ref):
    o_ref[...] = jnp.maximum(
        jnp.dot(f_ref[...], w_ref[...], preferred_element_type=jnp.float32)
        + b_ref[...], 0.0)


def _node_transform(features, W_trans, b_trans):
    return pl.pallas_call(
        _tc0_body,
        grid=(_N // _BM,),
        in_specs=[
            pl.BlockSpec((_BM, _H), lambda i: (i, 0)),
            pl.BlockSpec((_H, _H), lambda i: (0, 0)),
            pl.BlockSpec((1, _H), lambda i: (0, 0)),
        ],
        out_specs=pl.BlockSpec((_BM, _H), lambda i: (i, 0)),
        out_shape=jax.ShapeDtypeStruct((_N, _H), jnp.float32),
    )(features, W_trans, b_trans.reshape(1, _H))


def _steps_body(n_p, pmap, *refs):
    k = len(pmap)
    ps = refs[: 2 * n_p]
    d0, d1, cs, ws = refs[2 * n_p: 2 * n_p + 4]
    outs = refs[2 * n_p + 4:]
    deg = d0[0][:, 0:1] + d1[0][:, 0:1]
    inv = 1.0 / jnp.maximum(deg, 1.0)
    aggs = [(ps[2 * j][0] + ps[2 * j + 1][0]) * inv for j in range(n_p)]
    c = cs[...]
    for j in range(k):
        b = aggs[pmap[j]] * c[j][None, :]
        outs[j][...] = jnp.maximum(
            jnp.dot(b, ws[j], preferred_element_type=jnp.float32), 0.0)


def _gcn_steps(p_list, dcnt, cs, wsel, pmap):
    n_p = len(p_list)
    k = len(pmap)
    in_specs = []
    args = []
    for p in p_list:
        in_specs.append(pl.BlockSpec((1, _BM, _H), lambda i: (0, i, 0)))
        in_specs.append(pl.BlockSpec((1, _BM, _H), lambda i: (1, i, 0)))
        args += [p, p]
    in_specs.append(pl.BlockSpec((1, _BM, _H), lambda i: (0, i, 0)))
    in_specs.append(pl.BlockSpec((1, _BM, _H), lambda i: (1, i, 0)))
    args += [dcnt, dcnt]
    in_specs.append(pl.BlockSpec((k, _H), lambda i: (0, 0)))
    in_specs.append(pl.BlockSpec((k, _H, _H), lambda i: (0, 0, 0)))
    args += [cs, wsel]
    out = pl.pallas_call(
        functools.partial(_steps_body, n_p, pmap),
        grid=(_N // _BM,),
        in_specs=in_specs,
        out_specs=[pl.BlockSpec((_BM, _H), lambda i: (i, 0))] * k,
        out_shape=[jax.ShapeDtypeStruct((_N, _H), jnp.float32)] * k,
    )
    return out(*args)


def _fa_body(h0, h1, h2, v_ref, fc_ref, vr_ref, out):
    i = pl.program_id(0)

    @pl.when(i == 0)
    def _():
        out[...] = jnp.zeros((8, _H), jnp.float32)

    vals = []
    rsum = jnp.float32(0.0)
    fc = fc_ref[...]
    for j, h in enumerate((h0, h1, h2)):
        hb = h[...]
        z = jnp.tanh(jnp.dot(hb, v_ref[...], preferred_element_type=jnp.float32))
        vals.append(jnp.sum(z * fc[j][None, :]))
        r = jnp.dot(z, vr_ref[...], preferred_element_type=jnp.float32) - hb
        rsum = rsum + jnp.sum(r * r)
    z0 = jnp.float32(0.0)
    upd = jnp.stack([vals[0], vals[1], vals[2], rsum, z0, z0, z0, z0])
    out[...] += jnp.broadcast_to(upd[:, None], (8, _H))


def _fusion_stats(hs, V, fc, Vrev):
    return pl.pallas_call(
        _fa_body,
        grid=(_N // _BN,),
        in_specs=[pl.BlockSpec((_BN, _H), lambda i: (i, 0))] * 3
        + [
            pl.BlockSpec((_H, 64), lambda i: (0, 0)),
            pl.BlockSpec((3, 64), lambda i: (0, 0)),
            pl.BlockSpec((64, _H), lambda i: (0, 0)),
        ],
        out_specs=pl.BlockSpec((8, _H), lambda i: (0, 0)),
        out_shape=jax.ShapeDtypeStruct((8, _H), jnp.float32),
    )(hs[0], hs[1], hs[2], V, fc, Vrev)


def _beta(s_ref):
    s = [s_ref[j, 0] / _N for j in range(3)]
    m = jnp.maximum(jnp.maximum(s[0], s[1]), s[2])
    e = [jnp.exp(v - m) for v in s]
    t = e[0] + e[1] + e[2]
    return [v / t for v in e]


def _ortho(v_ref):
    v = v_ref[...]
    vtv = lax.dot_general(v, v, (((0,), (0,)), ((), ())),
                          preferred_element_type=jnp.float32)
    eye = jnp.eye(64, dtype=jnp.float32)
    d = vtv - eye
    return jnp.sum(d * d) / (64.0 * 64.0)


def _fb_body(su, si, hu0, hu1, hu2, hi0, hi1, hi2, vu, vi, ou, oi, ol):
    i = pl.program_id(0)
    bu = _beta(su)
    bi = _beta(si)
    ou[...] = bu[0] * hu0[...] + bu[1] * hu1[...] + bu[2] * hu2[...]
    oi[...] = bi[0] * hi0[...] + bi[1] * hi1[...] + bi[2] * hi2[...]

    @pl.when(i == 0)
    def _():
        denom = jnp.float32(_N * _H)
        loss = (su[3, 0] / denom + _ortho(vu)
                + si[3, 0] / denom + _ortho(vi))
        ol[...] = jnp.full((8, _H), loss, jnp.float32)


def _fusion_out(su, si, uo, io, V_user, V_item):
    small = lambda a, b: pl.BlockSpec((a, b), lambda i: (0, 0))
    return pl.pallas_call(
        _fb_body,
        grid=(_N // _BN,),
        in_specs=[small(8, _H), small(8, _H)]
        + [pl.BlockSpec((_BN, _H), lambda i: (i, 0))] * 6
        + [small(_H, 64), small(_H, 64)],
        out_specs=[pl.BlockSpec((_BN, _H), lambda i: (i, 0))] * 2
        + [small(8, _H)],
        out_shape=[jax.ShapeDtypeStruct((_N, _H), jnp.float32)] * 2
        + [jax.ShapeDtypeStruct((8, _H), jnp.float32)],
    )(su, si, uo[0], uo[1], uo[2], io[0], io[1], io[2], V_user, V_item)


def kernel(features, edge_index, W_trans, b_trans, comp_vecs, gcn_W,
           V_user, Vrev_user, fc_user, V_item, Vrev_item, fc_item):
    src = edge_index[0]
    dst = edge_index[1]

    c = [comp_vecs[j] for j in range(5)]
    w = [gcn_W[j] for j in range(3)]

    x = _node_transform(features, W_trans, b_trans)

    dcnt = _deg(dst)
    p_x = _agg(x, src, dst)
    h0, h1, h2, h4 = _gcn_steps(
        [p_x], dcnt,
        jnp.stack([c[0], c[1], c[2], c[4]]),
        jnp.stack([w[2], w[1], w[1], w[1]]),
        (0, 0, 0, 0))

    p0 = _agg(h0, src, dst)
    p1 = _agg(h1, src, dst)
    p2 = _agg(h2, src, dst)
    h01, h02, h10, h14, h23 = _gcn_steps(
        [p0, p1, p2], dcnt,
        jnp.stack([c[1], c[2], c[0], c[4], c[3]]),
        jnp.stack([w[1], w[1], w[2], w[1], w[2]]),
        (0, 0, 1, 1, 2))

    p02 = _agg(h02, src, dst)
    p14 = _agg(h14, src, dst)
    h023, h140 = _gcn_steps(
        [p02, p14], dcnt,
        jnp.stack([c[3], c[0]]),
        jnp.stack([w[2], w[2]]),
        (0, 1))

    p023 = _agg(h023, src, dst)
    (h0231,) = _gcn_steps(
        [p023], dcnt,
        jnp.stack([c[1]]),
        jnp.stack([w[1]]),
        (0,))

    uo = [h01, h0231, h4]
    io = [h10, h23, h140]

    su = _fusion_stats(uo, V_user, fc_user, Vrev_user)
    si = _fusion_stats(io, V_item, fc_item, Vrev_item)
    h_user, h_item, lbuf = _fusion_out(su, si, uo, io, V_user, V_item)
    return (h_user, h_item, lbuf[0, 0])
